# baseline (speedup 1.0000x reference)
import functools

import jax
import jax.numpy as jnp
from jax import lax
from jax.experimental import pallas as pl
from jax.experimental.pallas import tpu as pltpu
from jax.experimental.pallas import tpu_sc as plsc

N_ITEMS = 1000
N_PERSONS = 100000
BATCH = 16384
_NC = 2
_NS = 16
_NW = _NC * _NS
_BPW = BATCH // _NW
_HALF_LOG_2PI = 0.9189385332046727
_N_PARAMS = N_ITEMS + 4 * N_ITEMS + N_PERSONS


def _sp(x):
    return jnp.maximum(x, 0.0) + jnp.log(1.0 + jnp.exp(-jnp.abs(x)))


def _sig(x):
    return 1.0 / (1.0 + jnp.exp(-x))


def _sc_gather(a1, bb1, d01, d11, d21, t, item1, person1):
    mesh = plsc.VectorSubcoreMesh(core_axis_name="c", subcore_axis_name="s")
    out_type = jax.ShapeDtypeStruct((_NW, 6, _BPW), jnp.float32)
    scratch = [
        pltpu.VMEM((5120,), jnp.float32),
        pltpu.VMEM((_BPW,), jnp.int32),
        pltpu.VMEM((_BPW,), jnp.int32),
        pltpu.VMEM((6, _BPW), jnp.float32),
        pltpu.SemaphoreType.DMA,
        pltpu.SemaphoreType.DMA,
    ]

    @functools.partial(
        pl.kernel, mesh=mesh, out_type=out_type, scratch_types=scratch,
        compiler_params=pltpu.CompilerParams(
            use_tc_tiling_on_sc=False, needs_layout_passes=False))
    def k(a_h, bb_h, d0_h, d1_h, d2_h, t_h, item_h, person_h, out_h,
          pk, ii, ip, buf, sem, sem2):
        wid = lax.axis_index("s") * _NC + lax.axis_index("c")
        base = wid * _BPW
        pltpu.sync_copy(person_h.at[pl.ds(base, _BPW)], ip)
        pltpu.sync_copy(item_h.at[pl.ds(base, _BPW)], ii)
        copies = [
            pltpu.async_copy(t_h.at[ip.at[pl.ds(j * 128, 128)]],
                             buf.at[5, pl.ds(j * 128, 128)], sem)
            for j in range(_BPW // 128)
        ]
        tcopies = [
            pltpu.async_copy(src, pk.at[pl.ds(1024 * n, N_ITEMS)], sem2)
            for n, src in enumerate((a_h, bb_h, d0_h, d1_h, d2_h))
        ]
        for c in tcopies:
            c.wait()

        def body(i, _):
            sl = pl.ds(i * 16, 16)
            it = ii[sl]
            buf[0, sl] = plsc.load_gather(pk, [it])
            buf[1, sl] = plsc.load_gather(pk, [it + 1024])
            buf[2, sl] = plsc.load_gather(pk, [it + 2048])
            buf[3, sl] = plsc.load_gather(pk, [it + 3072])
            buf[4, sl] = plsc.load_gather(pk, [it + 4096])
            return 0

        lax.fori_loop(0, _BPW // 16, body, 0)
        for c in copies:
            c.wait()
        pltpu.sync_copy(buf, out_h.at[wid])

    return k(a1, bb1, d01, d11, d21, t, item1, person1)


def _final_body(a_ref, bb_ref, d0_ref, d1_ref, d2_ref, t_ref,
                g_ref, resp_ref, out_ref):
    a = _sp(a_ref[...])
    b0 = bb_ref[...]
    b1 = b0 + _sp(d0_ref[...])
    b2 = b1 + _sp(d1_ref[...])
    b3 = b2 + _sp(d2_ref[...])
    sq = (jnp.sum(a * a) + jnp.sum(b0 * b0 + b1 * b1 + b2 * b2 + b3 * b3)
          + jnp.sum(t_ref[...] ** 2))
    log_prior = -0.5 * sq - _HALF_LOG_2PI * _N_PARAMS

    ai = _sp(g_ref[0])
    gb0 = g_ref[1]
    gb1 = gb0 + _sp(g_ref[2])
    gb2 = gb1 + _sp(g_ref[3])
    gb3 = gb2 + _sp(g_ref[4])
    gt = g_ref[5]
    r = resp_ref[...]
    bu = jnp.where(r == 2, gb0, jnp.where(r == 3, gb1,
                   jnp.where(r == 4, gb2, gb3)))
    bl = jnp.where(r == 1, gb0, jnp.where(r == 2, gb1,
                   jnp.where(r == 3, gb2, gb3)))
    upper = jnp.where(r == 1, 1.0, _sig(ai * (gt - bu)))
    lower = jnp.where(r == 5, 0.0, _sig(ai * (gt - bl)))
    ll = jnp.sum(jnp.log(upper - lower + 1e-10))

    out_ref[0, 0] = -(ll + log_prior * (BATCH / 1e6))


def kernel(a_, b_base_, b_diff_, t, indices):
    item1 = indices[:, 0]
    person1 = indices[:, 1]
    resp1 = indices[:, 2]
    bb1 = b_base_[:, 0]
    d01 = b_diff_[:, 0]
    d11 = b_diff_[:, 1]
    d21 = b_diff_[:, 2]

    g = _sc_gather(a_, bb1, d01, d11, d21, t, item1, person1)
    g6 = g.transpose(1, 0, 2).reshape(6, 128, 128)

    pad_neg = lambda x: jnp.pad(x, (0, 24), constant_values=-100.0).reshape(8, 128)
    pad_bb = jnp.pad(bb1, (0, 24)).reshape(8, 128)
    pad_t = jnp.pad(t, (0, 352)).reshape(784, 128)
    out = pl.pallas_call(
        _final_body,
        out_shape=jax.ShapeDtypeStruct((1, 1), jnp.float32),
        out_specs=pl.BlockSpec(memory_space=pltpu.SMEM),
    )(pad_neg(a_), pad_bb, pad_neg(d01), pad_neg(d11), pad_neg(d21), pad_t,
      g6, resp1.reshape(128, 128))
    return out[0, 0]


# speedup vs baseline: 1.0254x; 1.0254x over previous
import functools

import jax
import jax.numpy as jnp
from jax import lax
from jax.experimental import pallas as pl
from jax.experimental.pallas import tpu as pltpu
from jax.experimental.pallas import tpu_sc as plsc

N_ITEMS = 1000
N_PERSONS = 100000
BATCH = 16384
_NC = 2
_NS = 16
_NW = _NC * _NS
_BPW = BATCH // _NW
_HALF_LOG_2PI = 0.9189385332046727
_N_PARAMS = N_ITEMS + 4 * N_ITEMS + N_PERSONS


def _sp(x):
    return jnp.maximum(x, 0.0) + jnp.log(1.0 + jnp.exp(-jnp.abs(x)))


def _sig(x):
    return 1.0 / (1.0 + jnp.exp(-x))


def _table_body(raw_ref, out_ref):
    raw = raw_ref[...]
    a = _sp(raw[0:8])
    b0 = raw[8:16]
    b1 = b0 + _sp(raw[16:24])
    b2 = b1 + _sp(raw[24:32])
    b3 = b2 + _sp(raw[32:40])
    out_ref[pl.ds(0, 8), :] = a
    out_ref[pl.ds(8, 8), :] = b0
    out_ref[pl.ds(16, 8), :] = b1
    out_ref[pl.ds(24, 8), :] = b2
    out_ref[pl.ds(32, 8), :] = b3


def _sc_gather(table, t, item1, person1, resp1):
    mesh = plsc.VectorSubcoreMesh(core_axis_name="c", subcore_axis_name="s")
    out_types = [jax.ShapeDtypeStruct((BATCH,), jnp.float32)
                 for _ in range(4)]
    scratch = (
        [pltpu.VMEM((40, 128), jnp.float32)]
        + [pltpu.VMEM((_BPW,), jnp.int32) for _ in range(3)]
        + [pltpu.VMEM((_BPW,), jnp.float32) for _ in range(4)]
        + [pltpu.SemaphoreType.DMA]
    )

    @functools.partial(
        pl.kernel, mesh=mesh, out_type=out_types, scratch_types=scratch,
        compiler_params=pltpu.CompilerParams(
            use_tc_tiling_on_sc=False, needs_layout_passes=False))
    def k(table_h, t_h, item_h, person_h, resp_h,
          oa, obu, obl, ot,
          pk, ii, ip, ir, ba, bu, bl, bt, sem):
        wid = lax.axis_index("s") * _NC + lax.axis_index("c")
        base = wid * _BPW
        pltpu.sync_copy(item_h.at[pl.ds(base, _BPW)], ii)
        pltpu.sync_copy(person_h.at[pl.ds(base, _BPW)], ip)
        pltpu.sync_copy(resp_h.at[pl.ds(base, _BPW)], ir)
        copies = [
            pltpu.async_copy(t_h.at[ip.at[pl.ds(j * 128, 128)]],
                             bt.at[pl.ds(j * 128, 128)], sem)
            for j in range(_BPW // 128)
        ]
        pltpu.sync_copy(table_h, pk)

        def body(i, _):
            sl = pl.ds(i * 16, 16)
            it = ii[sl]
            r = ir[sl]
            ihi = lax.shift_right_logical(it, 7)
            ilo = lax.bitwise_and(it, 127)
            u = jnp.maximum(r - 2, 0)
            l = jnp.minimum(r - 1, 3)
            ba[sl] = plsc.load_gather(pk, [ihi, ilo])
            bu[sl] = plsc.load_gather(
                pk, [lax.shift_left(u + 1, 3) + ihi, ilo])
            bl[sl] = plsc.load_gather(
                pk, [lax.shift_left(l + 1, 3) + ihi, ilo])
            return 0

        lax.fori_loop(0, _BPW // 16, body, 0)
        for c in copies:
            c.wait()
        pltpu.sync_copy(ba, oa.at[pl.ds(base, _BPW)])
        pltpu.sync_copy(bu, obu.at[pl.ds(base, _BPW)])
        pltpu.sync_copy(bl, obl.at[pl.ds(base, _BPW)])
        pltpu.sync_copy(bt, ot.at[pl.ds(base, _BPW)])

    return k(table, t, item1, person1, resp1)


def _final_body(table_ref, t_ref, ga_ref, gbu_ref, gbl_ref, gt_ref, resp_ref,
                out_ref):
    tab = table_ref[...]
    tv = t_ref[...]
    sq = jnp.sum(tab * tab) + jnp.sum(tv * tv)
    log_prior = -0.5 * sq - _HALF_LOG_2PI * _N_PARAMS

    ai = ga_ref[...]
    gt = gt_ref[...]
    r = resp_ref[...]
    upper = jnp.where(r == 1, 1.0, _sig(ai * (gt - gbu_ref[...])))
    lower = jnp.where(r == 5, 0.0, _sig(ai * (gt - gbl_ref[...])))
    ll = jnp.sum(jnp.log(upper - lower + 1e-10))

    out_ref[0, 0] = -(ll + log_prior * (BATCH / 1e6))


def kernel(a_, b_base_, b_diff_, t, indices):
    item1 = indices[:, 0]
    person1 = indices[:, 1]
    resp1 = indices[:, 2]

    pad_neg = lambda x: jnp.pad(x, (0, 24), constant_values=-100.0)
    stacked = jnp.concatenate([
        pad_neg(a_).reshape(8, 128),
        jnp.pad(b_base_[:, 0], (0, 24)).reshape(8, 128),
        pad_neg(b_diff_[:, 0]).reshape(8, 128),
        pad_neg(b_diff_[:, 1]).reshape(8, 128),
        pad_neg(b_diff_[:, 2]).reshape(8, 128),
    ], axis=0)

    table = pl.pallas_call(
        _table_body,
        out_shape=jax.ShapeDtypeStruct((40, 128), jnp.float32),
    )(stacked)

    ga, gbu, gbl, gt = _sc_gather(table, t, item1, person1, resp1)

    sq128 = lambda x: x.reshape(128, 128)
    pad_t = jnp.pad(t, (0, 352)).reshape(784, 128)
    out = pl.pallas_call(
        _final_body,
        out_shape=jax.ShapeDtypeStruct((1, 1), jnp.float32),
        out_specs=pl.BlockSpec(memory_space=pltpu.SMEM),
    )(table, pad_t, sq128(ga), sq128(gbu), sq128(gbl), sq128(gt),
      sq128(resp1))
    return out[0, 0]
